# Initial kernel scaffold; baseline (speedup 1.0000x reference)
import jax
import jax.numpy as jnp
from jax.experimental import pallas as pl


def kernel(images, domains, W, b):
    raise NotImplementedError("write your pallas kernel here")


# speedup vs baseline: 1.5546x; 1.5546x over previous
import functools

import jax
import jax.numpy as jnp
from jax import lax
from jax.experimental import pallas as pl
from jax.experimental.pallas import tpu as pltpu
from jax.experimental.pallas import tpu_sc as plsc

NUM_EXPERTS = 8
BATCH = 4096
D_IN = 2048
D_OUT = 2048
TILE_M = 128
PADDED = BATCH + NUM_EXPERTS * TILE_M
NUM_TILES = PADDED // TILE_M

_NC, _NS = 2, 16
_NW = _NC * _NS


@functools.lru_cache(maxsize=None)
def _sc_mesh():
    return plsc.VectorSubcoreMesh(
        core_axis_name="c", subcore_axis_name="s", num_cores=_NC, num_subcores=_NS
    )


def _routing(domains):
    d = domains.astype(jnp.int32)
    onehot = (d[:, None] == jnp.arange(NUM_EXPERTS, dtype=jnp.int32)[None, :])
    onehot = onehot.astype(jnp.int32)
    rank = jnp.sum(jnp.cumsum(onehot, axis=0) * onehot, axis=1) - 1
    counts = jnp.sum(onehot, axis=0)
    padded_counts = ((counts + TILE_M - 1) // TILE_M) * TILE_M
    ends = jnp.cumsum(padded_counts)
    starts = ends - padded_counts
    dest = starts[d] + rank
    gather_src = jnp.zeros((PADDED,), jnp.int32).at[dest].set(
        jnp.arange(BATCH, dtype=jnp.int32)
    )
    tile_ids = jnp.arange(NUM_TILES, dtype=jnp.int32) * TILE_M
    tile_expert = jnp.clip(
        jnp.searchsorted(ends, tile_ids, side="right"), 0, NUM_EXPERTS - 1
    ).astype(jnp.int32)
    return gather_src, dest, tile_expert


@functools.lru_cache(maxsize=None)
def _make_sc_gather(B, D):
    rpw = B // _NW
    CH = 32
    assert rpw % CH == 0
    nch = rpw // CH

    @functools.partial(
        pl.kernel,
        out_type=jax.ShapeDtypeStruct((B, D), jnp.float32),
        mesh=_sc_mesh(),
        scratch_types=[
            pltpu.VMEM((CH,), jnp.int32),
            pltpu.VMEM((CH, D), jnp.float32),
            pltpu.SemaphoreType.DMA,
        ],
    )
    def gather_k(table_hbm, idx_hbm, out_hbm, idx_v, rows_v, sem):
        wid = lax.axis_index("s") * _NC + lax.axis_index("c")
        base = wid * rpw
        for c in range(nch):
            off = base + c * CH
            pltpu.sync_copy(idx_hbm.at[pl.ds(off, CH)], idx_v)
            pltpu.async_copy(table_hbm.at[idx_v], rows_v, sem).wait()
            pltpu.sync_copy(rows_v, out_hbm.at[pl.ds(off, CH)])

    return gather_k


def _mm_body(te_ref, x_ref, w_ref, b_ref, y_ref):
    del te_ref
    y_ref[...] = (
        jnp.dot(x_ref[...], w_ref[0], preferred_element_type=jnp.float32)
        + b_ref[0]
    )


def _grouped_matmul(x_sorted, W, b, tile_expert):
    grid_spec = pltpu.PrefetchScalarGridSpec(
        num_scalar_prefetch=1,
        grid=(NUM_TILES,),
        in_specs=[
            pl.BlockSpec((TILE_M, D_IN), lambda i, te: (i, 0)),
            pl.BlockSpec((1, D_IN, D_OUT), lambda i, te: (te[i], 0, 0)),
            pl.BlockSpec((1, 1, D_OUT), lambda i, te: (te[i], 0, 0)),
        ],
        out_specs=pl.BlockSpec((TILE_M, D_OUT), lambda i, te: (i, 0)),
    )
    return pl.pallas_call(
        _mm_body,
        grid_spec=grid_spec,
        out_shape=jax.ShapeDtypeStruct((PADDED, D_OUT), jnp.float32),
    )(tile_expert, x_sorted, W, b.reshape(NUM_EXPERTS, 1, D_OUT))


def kernel(images, domains, W, b):
    gather_src, dest, tile_expert = _routing(domains)
    x_sorted = _make_sc_gather(PADDED, D_IN)(images, gather_src)
    y_sorted = _grouped_matmul(x_sorted, W, b, tile_expert)
    outputs = _make_sc_gather(BATCH, D_OUT)(y_sorted, dest)
    return outputs
